# baseline (speedup 1.0000x reference)
import functools

import jax
import jax.numpy as jnp
from jax import lax
from jax.experimental import pallas as pl
from jax.experimental.pallas import tpu as pltpu
from jax.experimental.pallas import tpu_sc as plsc

_NC = 2
_NS = 16
_NW = _NC * _NS
_NPAD = 10240


def _proj_body(x_ref, x2_ref, w_ref, brow_ref, od_ref, os_ref, ot_ref, *,
               act, wd, thalf):
    xb = x_ref[...]
    if x2_ref is not None:
        xb = xb + x2_ref[...]
    if act:
        xb = jnp.maximum(xb, 0.0)
    res = jnp.dot(xb, w_ref[...], preferred_element_type=jnp.float32)
    od_ref[...] = res[:, :wd]
    os_ref[...] = res[:, wd:2 * wd]
    t = res[:, 2 * wd:] + brow_ref[...]
    ot_ref[...] = t * 0.5 if thalf else t


def _node_proj(hins, wcat, brow, act, wd, wt, thalf, rb=2000):
    n, f = 10000, hins[0].shape[1]
    k = wcat.shape[1]
    two = len(hins) == 2
    body = functools.partial(_proj_body, act=act, wd=wd, thalf=thalf)
    if not two:
        body = functools.partial(lambda b, x, w, br, od, os_, ot:
                                 b(x, None, w, br, od, os_, ot), body)
    in_specs = [pl.BlockSpec((rb, f), lambda i: (i, 0))]
    if two:
        in_specs.append(pl.BlockSpec((rb, f), lambda i: (i, 0)))
    in_specs += [
        pl.BlockSpec((f, k), lambda i: (0, 0)),
        pl.BlockSpec((1, wt), lambda i: (0, 0)),
    ]
    return pl.pallas_call(
        body,
        grid=(n // rb,),
        in_specs=in_specs,
        out_specs=[
            pl.BlockSpec((rb, wd), lambda i: (i, 0)),
            pl.BlockSpec((rb, wd), lambda i: (i, 0)),
            pl.BlockSpec((rb, wt), lambda i: (i, 0)),
        ],
        out_shape=[
            jax.ShapeDtypeStruct((n, wd), jnp.float32),
            jax.ShapeDtypeStruct((n, wd), jnp.float32),
            jax.ShapeDtypeStruct((n, wt), jnp.float32),
        ],
    )(*hins, wcat, brow)


def _gate_body(gd_ref, gs_ref, ea_ref, wfe_ref, wse_ref, bf_ref, bs_ref,
               o_ref, *, h):
    gd = gd_ref[...]
    gs = gs_ref[...]
    ea = ea_ref[...]
    f = gd[:, :h] + gs[:, :h] + bf_ref[...] + jnp.dot(
        ea, wfe_ref[...], preferred_element_type=jnp.float32)
    s = gd[:, h:] + gs[:, h:] + bs_ref[...] + jnp.dot(
        ea, wse_ref[...], preferred_element_type=jnp.float32)
    o_ref[...] = jax.nn.sigmoid(f) * jax.nn.softplus(s)


def _gate(gd, gs, ea, wfe, wse, bf, bs, h, eb=2000):
    e = gd.shape[0]
    d = ea.shape[1]
    return pl.pallas_call(
        functools.partial(_gate_body, h=h),
        grid=(e // eb,),
        in_specs=[
            pl.BlockSpec((eb, 2 * h), lambda i: (i, 0)),
            pl.BlockSpec((eb, 2 * h), lambda i: (i, 0)),
            pl.BlockSpec((eb, d), lambda i: (i, 0)),
            pl.BlockSpec((d, h), lambda i: (0, 0)),
            pl.BlockSpec((d, h), lambda i: (0, 0)),
            pl.BlockSpec((1, h), lambda i: (0, 0)),
            pl.BlockSpec((1, h), lambda i: (0, 0)),
        ],
        out_specs=pl.BlockSpec((eb, h), lambda i: (i, 0)),
        out_shape=jax.ShapeDtypeStruct((e, h), jnp.float32),
    )(gd, gs, ea, wfe, wse, bf, bs)


def _cls_body(h_ref, g_ref, brow_ref, wc_ref, bc_ref, o_ref):
    inv = 1.0 / jnp.sqrt(1.0 + 1e-5)
    wce = wc_ref[...] * (g_ref[...] * inv)
    off = jnp.dot(brow_ref[...], wc_ref[...],
                  preferred_element_type=jnp.float32)
    o_ref[...] = jnp.dot(h_ref[...], wce,
                         preferred_element_type=jnp.float32) + off + bc_ref[...]


def _classifier(h, gamma_col, beta_row, wc, bc_row, rb=2000):
    o = h.shape[1]
    n = 10000
    c = wc.shape[1]
    return pl.pallas_call(
        _cls_body,
        grid=(n // rb,),
        in_specs=[
            pl.BlockSpec((rb, o), lambda i: (i, 0)),
            pl.BlockSpec((o, 1), lambda i: (0, 0)),
            pl.BlockSpec((1, o), lambda i: (0, 0)),
            pl.BlockSpec((o, c), lambda i: (0, 0)),
            pl.BlockSpec((1, c), lambda i: (0, 0)),
        ],
        out_specs=pl.BlockSpec((rb, c), lambda i: (i, 0)),
        out_shape=jax.ShapeDtypeStruct((n, c), jnp.float32),
    )(h, gamma_col, beta_row, wc, bc_row)


def _make_gather(e_total, width, kchunk=80):
    epw = e_total // _NW
    nchunks = epw // kchunk
    mesh = plsc.VectorSubcoreMesh(core_axis_name="c", subcore_axis_name="s")

    @functools.partial(
        pl.kernel, mesh=mesh,
        out_type=[
            jax.ShapeDtypeStruct((e_total, width), jnp.float32),
            jax.ShapeDtypeStruct((e_total, width), jnp.float32),
        ],
        scratch_types=(
            [pltpu.VMEM((kchunk,), jnp.int32)] * 6
            + [pltpu.VMEM((kchunk, width), jnp.float32)] * 6
            + [pltpu.SemaphoreType.DMA] * 9
        ),
    )
    def gather_k(tbl_d, tbl_s, dst, src, out_d, out_s,
                 idx_d0, idx_s0, idx_d1, idx_s1, idx_d2, idx_s2,
                 buf_d0, buf_s0, buf_d1, buf_s1, buf_d2, buf_s2,
                 sem_d0, sem_s0, sem_d1, sem_s1, sem_d2, sem_s2,
                 sem_w0, sem_w1, sem_w2):
        wid = lax.axis_index("s") * _NC + lax.axis_index("c")
        base = wid * epw
        slots = ((idx_d0, idx_s0, buf_d0, buf_s0, sem_d0, sem_s0, sem_w0),
                 (idx_d1, idx_s1, buf_d1, buf_s1, sem_d1, sem_s1, sem_w1),
                 (idx_d2, idx_s2, buf_d2, buf_s2, sem_d2, sem_s2, sem_w2))

        def load_and_fire(ci, slot):
            idx_d, idx_s, buf_d, buf_s, sem_d, sem_s, _ = slot
            eb = base + ci * kchunk
            pltpu.sync_copy(dst.at[pl.ds(eb, kchunk)], idx_d)
            pltpu.sync_copy(src.at[pl.ds(eb, kchunk)], idx_s)
            pltpu.async_copy(tbl_d.at[idx_d], buf_d, sem_d)
            pltpu.async_copy(tbl_s.at[idx_s], buf_s, sem_s)

        def wait_writes(ci, slot):
            _, _, buf_d, buf_s, _, _, sem_w = slot
            eb = base + ci * kchunk
            pltpu.make_async_copy(buf_d, out_d.at[pl.ds(eb, kchunk)],
                                  sem_w).wait()
            pltpu.make_async_copy(buf_s, out_s.at[pl.ds(eb, kchunk)],
                                  sem_w).wait()

        load_and_fire(0, slots[0])
        load_and_fire(1, slots[1])

        def body(ci, carry):
            def per_slot(cur, nxt2):
                idx_d, idx_s, buf_d, buf_s, sem_d, sem_s, sem_w = cur
                eb = base + ci * kchunk
                pltpu.make_async_copy(tbl_d.at[idx_d], buf_d, sem_d).wait()
                pltpu.make_async_copy(tbl_s.at[idx_s], buf_s, sem_s).wait()
                pltpu.async_copy(buf_d, out_d.at[pl.ds(eb, kchunk)], sem_w)
                pltpu.async_copy(buf_s, out_s.at[pl.ds(eb, kchunk)], sem_w)

                @pl.when(ci + 2 < nchunks)
                def _():
                    @pl.when(ci >= 1)
                    def _():
                        wait_writes(ci - 1, nxt2)

                    load_and_fire(ci + 2, nxt2)

            for p in range(3):
                @pl.when(lax.rem(ci, 3) == p)
                def _(p=p):
                    per_slot(slots[p], slots[(p + 2) % 3])

            return carry

        lax.fori_loop(0, nchunks, body, 0)
        for ci in range(max(nchunks - 3, 0), nchunks):
            wait_writes(ci, slots[ci % 3])

    return gather_k


def _make_scatter_esplit(e_total, h_out, kchunk=80):
    epw = e_total // _NW
    nchunks = epw // kchunk
    rows_pt = _NPAD // _NS
    mesh = plsc.VectorSubcoreMesh(core_axis_name="c", subcore_axis_name="s")

    @functools.partial(
        pl.kernel, mesh=mesh,
        out_type=[
            jax.ShapeDtypeStruct((_NPAD, h_out), jnp.float32),
            jax.ShapeDtypeStruct((_NPAD, h_out), jnp.float32),
        ],
        scratch_types=[
            pltpu.VMEM((kchunk,), jnp.int32),
            pltpu.VMEM((kchunk,), jnp.int32),
            pltpu.VMEM((kchunk, h_out), jnp.float32),
            pltpu.VMEM((kchunk, h_out), jnp.float32),
            pltpu.VMEM_SHARED((_NPAD, h_out), jnp.float32),
            pltpu.SemaphoreType.DMA,
            pltpu.SemaphoreType.DMA,
        ],
    )
    def scatter_k(m_a, m_b, dstn, t_half, out0, out1,
                  idx0, idx1, buf0, buf1, acc, sem0, sem1):
        cid = lax.axis_index("c")
        sid = lax.axis_index("s")
        r0 = sid * rows_pt
        pltpu.sync_copy(t_half.at[pl.ds(r0, rows_pt)],
                        acc.at[pl.ds(r0, rows_pt)])
        plsc.subcore_barrier()
        base = sid * epw
        slots = ((idx0, buf0, sem0), (idx1, buf1, sem1))

        def run_half(m, dst_off):
            def load(ci, slot):
                idx_v, buf, sem = slot
                eb = base + ci * kchunk
                pltpu.async_copy(dstn.at[pl.ds(dst_off + eb, kchunk)], idx_v,
                                 sem)
                pltpu.async_copy(m.at[pl.ds(eb, kchunk)], buf, sem)

            load(0, slots[0])

            def chunk(ci, carry):
                def per_slot(cur, nxt):
                    idx_v, buf, sem = cur
                    eb = base + ci * kchunk
                    pltpu.make_async_copy(dstn.at[pl.ds(dst_off + eb, kchunk)],
                                          idx_v, sem).wait()
                    pltpu.make_async_copy(m.at[pl.ds(eb, kchunk)], buf,
                                          sem).wait()

                    @pl.when(ci + 1 < nchunks)
                    def _():
                        load(ci + 1, nxt)

                    pltpu.sync_copy(buf, acc.at[idx_v], add=True)

                @pl.when(lax.rem(ci, 2) == 0)
                def _():
                    per_slot(slots[0], slots[1])

                @pl.when(lax.rem(ci, 2) == 1)
                def _():
                    per_slot(slots[1], slots[0])

                return carry

            lax.fori_loop(0, nchunks, chunk, 0)

        @pl.when(cid == 0)
        def _():
            run_half(m_a, 0)

        @pl.when(cid == 1)
        def _():
            run_half(m_b, e_total // 2)

        plsc.subcore_barrier()

        @pl.when(cid == 0)
        def _():
            pltpu.sync_copy(acc.at[pl.ds(r0, rows_pt)],
                            out0.at[pl.ds(r0, rows_pt)])

        @pl.when(cid == 1)
        def _():
            pltpu.sync_copy(acc.at[pl.ds(r0, rows_pt)],
                            out1.at[pl.ds(r0, rows_pt)])

    return scatter_k


def _make_scatter_csplit(e_total, h_out, kchunk=80):
    hh = h_out // _NC
    epw = e_total // (2 * _NS)
    nchunks = epw // kchunk
    rows_pt = _NPAD // _NS
    mesh = plsc.VectorSubcoreMesh(core_axis_name="c", subcore_axis_name="s")

    @functools.partial(
        pl.kernel, mesh=mesh,
        out_type=jax.ShapeDtypeStruct((_NPAD, h_out), jnp.float32),
        scratch_types=[
            pltpu.VMEM((kchunk,), jnp.int32),
            pltpu.VMEM((kchunk,), jnp.int32),
            pltpu.VMEM((kchunk, hh), jnp.float32),
            pltpu.VMEM((kchunk, hh), jnp.float32),
            pltpu.VMEM_SHARED((_NPAD, hh), jnp.float32),
            pltpu.SemaphoreType.DMA,
            pltpu.SemaphoreType.DMA,
        ],
    )
    def scatter_k(m_a, m_b, dstn, t_init, out,
                  idx0, idx1, buf0, buf1, acc, sem0, sem1):
        cid = lax.axis_index("c")
        sid = lax.axis_index("s")
        col0 = cid * hh
        r0 = sid * rows_pt
        pltpu.sync_copy(t_init.at[pl.ds(r0, rows_pt), pl.ds(col0, hh)],
                        acc.at[pl.ds(r0, rows_pt)])
        plsc.subcore_barrier()
        base = sid * epw
        slots = ((idx0, buf0, sem0), (idx1, buf1, sem1))

        def run_half(m, dst_off):
            def load(ci, slot):
                idx_v, buf, sem = slot
                eb = base + ci * kchunk
                pltpu.async_copy(dstn.at[pl.ds(dst_off + eb, kchunk)], idx_v,
                                 sem)
                pltpu.async_copy(m.at[pl.ds(eb, kchunk), pl.ds(col0, hh)],
                                 buf, sem)

            load(0, slots[0])

            def chunk(ci, carry):
                def per_slot(cur, nxt):
                    idx_v, buf, sem = cur
                    eb = base + ci * kchunk
                    pltpu.make_async_copy(dstn.at[pl.ds(dst_off + eb, kchunk)],
                                          idx_v, sem).wait()
                    pltpu.make_async_copy(m.at[pl.ds(eb, kchunk),
                                               pl.ds(col0, hh)], buf,
                                          sem).wait()

                    @pl.when(ci + 1 < nchunks)
                    def _():
                        load(ci + 1, nxt)

                    pltpu.sync_copy(buf, acc.at[idx_v], add=True)

                @pl.when(lax.rem(ci, 2) == 0)
                def _():
                    per_slot(slots[0], slots[1])

                @pl.when(lax.rem(ci, 2) == 1)
                def _():
                    per_slot(slots[1], slots[0])

                return carry

            lax.fori_loop(0, nchunks, chunk, 0)

        run_half(m_a, 0)
        run_half(m_b, e_total // 2)
        plsc.subcore_barrier()
        pltpu.sync_copy(acc.at[pl.ds(r0, rows_pt)],
                        out.at[pl.ds(r0, rows_pt), pl.ds(col0, hh)])

    return scatter_k


def _cgconv_layer(hins, dst, src, edge_attr, Wf, bf, Ws, bs, Wt, bt, act,
                  colsplit):
    fin = hins[0].shape[1]
    e = dst.shape[0]
    h = Wt.shape[1]
    wd_ = jnp.concatenate([Wf[:fin], Ws[:fin]], axis=1)
    ws_ = jnp.concatenate([Wf[fin:2 * fin], Ws[fin:2 * fin]], axis=1)
    wcat = jnp.concatenate([wd_, ws_, Wt], axis=1)
    wfe = Wf[2 * fin:]
    wse = Ws[2 * fin:]

    tbl_d, tbl_s, t_self = _node_proj(hins, wcat, bt.reshape(1, h), act,
                                      wd=2 * h, wt=h, thalf=not colsplit)
    t_pad = jnp.pad(t_self, ((0, _NPAD - t_self.shape[0]), (0, 0)))
    eh = e // 2
    bfr, bsr = bf.reshape(1, h), bs.reshape(1, h)
    gath = _make_gather(eh, 2 * h, kchunk=40)
    gd_a, gs_a = gath(tbl_d, tbl_s, dst[:eh], src[:eh])
    gd_b, gs_b = gath(tbl_d, tbl_s, dst[eh:], src[eh:])
    m_a = _gate(gd_a, gs_a, edge_attr[:eh], wfe, wse, bfr, bsr, h)
    m_b = _gate(gd_b, gs_b, edge_attr[eh:], wfe, wse, bfr, bsr, h)
    if colsplit:
        return (_make_scatter_csplit(e, h)(m_a, m_b, dst, t_pad),)
    return _make_scatter_esplit(e, h)(m_a, m_b, dst, t_pad)


def kernel(x, edge_index, edge_attr,
           Wf1, bf1, Ws1, bs1, Wt1, bt1,
           Wf2, bf2, Ws2, bs2, Wt2, bt2,
           Wf3, bf3, Ws3, bs3, Wt3, bt3,
           bn_gamma, bn_beta, Wc, bc):
    src = edge_index[0]
    dst = edge_index[1]
    h1 = _cgconv_layer((x,), dst, src, edge_attr,
                       Wf1, bf1, Ws1, bs1, Wt1, bt1,
                       act=False, colsplit=False)
    h2 = _cgconv_layer(h1, dst, src, edge_attr,
                       Wf2, bf2, Ws2, bs2, Wt2, bt2,
                       act=True, colsplit=False)
    h3 = _cgconv_layer(h2, dst, src, edge_attr,
                       Wf3, bf3, Ws3, bs3, Wt3, bt3,
                       act=True, colsplit=True)[0]
    o = h3.shape[1]
    c = Wc.shape[1]
    return _classifier(h3, bn_gamma.reshape(o, 1), bn_beta.reshape(1, o),
                       Wc, bc.reshape(1, c))


# speedup vs baseline: 1.0759x; 1.0759x over previous
import functools

import jax
import jax.numpy as jnp
from jax import lax
from jax.experimental import pallas as pl
from jax.experimental.pallas import tpu as pltpu
from jax.experimental.pallas import tpu_sc as plsc

_NC = 2
_NS = 16
_NW = _NC * _NS
_NPAD = 10240


def _proj_body(x_ref, x2_ref, w_ref, brow_ref, od_ref, os_ref, ot_ref, *,
               act, wd, thalf):
    xb = x_ref[...]
    if x2_ref is not None:
        xb = xb + x2_ref[...]
    if act:
        xb = jnp.maximum(xb, 0.0)
    res = jnp.dot(xb, w_ref[...], preferred_element_type=jnp.float32)
    od_ref[...] = res[:, :wd].astype(jnp.bfloat16)
    os_ref[...] = res[:, wd:2 * wd].astype(jnp.bfloat16)
    t = res[:, 2 * wd:] + brow_ref[...]
    ot_ref[...] = t * 0.5 if thalf else t


def _node_proj(hins, wcat, brow, act, wd, wt, thalf, rb=2000):
    n, f = 10000, hins[0].shape[1]
    k = wcat.shape[1]
    two = len(hins) == 2
    body = functools.partial(_proj_body, act=act, wd=wd, thalf=thalf)
    if not two:
        body = functools.partial(lambda b, x, w, br, od, os_, ot:
                                 b(x, None, w, br, od, os_, ot), body)
    in_specs = [pl.BlockSpec((rb, f), lambda i: (i, 0))]
    if two:
        in_specs.append(pl.BlockSpec((rb, f), lambda i: (i, 0)))
    in_specs += [
        pl.BlockSpec((f, k), lambda i: (0, 0)),
        pl.BlockSpec((1, wt), lambda i: (0, 0)),
    ]
    return pl.pallas_call(
        body,
        grid=(n // rb,),
        in_specs=in_specs,
        out_specs=[
            pl.BlockSpec((rb, wd), lambda i: (i, 0)),
            pl.BlockSpec((rb, wd), lambda i: (i, 0)),
            pl.BlockSpec((rb, wt), lambda i: (i, 0)),
        ],
        out_shape=[
            jax.ShapeDtypeStruct((n, wd), jnp.bfloat16),
            jax.ShapeDtypeStruct((n, wd), jnp.bfloat16),
            jax.ShapeDtypeStruct((n, wt), jnp.float32),
        ],
    )(*hins, wcat, brow)


def _unpack_pair(raw):
    f = jax.lax.bitcast_convert_type(raw << 16, jnp.float32)
    s = jax.lax.bitcast_convert_type(raw & jnp.int32(-65536), jnp.float32)
    return f, s


def _gate_body(gd_ref, gs_ref, ea_ref, wfe_ref, wse_ref, bf_ref, bs_ref,
               o_ref, *, h):
    fd, sd = _unpack_pair(gd_ref[...])
    fs, ss = _unpack_pair(gs_ref[...])
    ea = ea_ref[...]
    f = fd + fs + bf_ref[...] + jnp.dot(
        ea, wfe_ref[...], preferred_element_type=jnp.float32)
    s = sd + ss + bs_ref[...] + jnp.dot(
        ea, wse_ref[...], preferred_element_type=jnp.float32)
    o_ref[...] = jax.nn.sigmoid(f) * jax.nn.softplus(s)


def _gate(gd, gs, ea, wfe, wse, bf, bs, h, eb=2000):
    e = gd.shape[0]
    d = ea.shape[1]
    return pl.pallas_call(
        functools.partial(_gate_body, h=h),
        grid=(e // eb,),
        in_specs=[
            pl.BlockSpec((eb, h), lambda i: (i, 0)),
            pl.BlockSpec((eb, h), lambda i: (i, 0)),
            pl.BlockSpec((eb, d), lambda i: (i, 0)),
            pl.BlockSpec((d, h), lambda i: (0, 0)),
            pl.BlockSpec((d, h), lambda i: (0, 0)),
            pl.BlockSpec((1, h), lambda i: (0, 0)),
            pl.BlockSpec((1, h), lambda i: (0, 0)),
        ],
        out_specs=pl.BlockSpec((eb, h), lambda i: (i, 0)),
        out_shape=jax.ShapeDtypeStruct((e, h), jnp.float32),
    )(gd, gs, ea, wfe, wse, bf, bs)


def _cls_body(h_ref, g_ref, brow_ref, wc_ref, bc_ref, o_ref):
    inv = 1.0 / jnp.sqrt(1.0 + 1e-5)
    wce = wc_ref[...] * (g_ref[...] * inv)
    off = jnp.dot(brow_ref[...], wc_ref[...],
                  preferred_element_type=jnp.float32)
    o_ref[...] = jnp.dot(h_ref[...], wce,
                         preferred_element_type=jnp.float32) + off + bc_ref[...]


def _classifier(h, gamma_col, beta_row, wc, bc_row, rb=2000):
    o = h.shape[1]
    n = 10000
    c = wc.shape[1]
    return pl.pallas_call(
        _cls_body,
        grid=(n // rb,),
        in_specs=[
            pl.BlockSpec((rb, o), lambda i: (i, 0)),
            pl.BlockSpec((o, 1), lambda i: (0, 0)),
            pl.BlockSpec((1, o), lambda i: (0, 0)),
            pl.BlockSpec((o, c), lambda i: (0, 0)),
            pl.BlockSpec((1, c), lambda i: (0, 0)),
        ],
        out_specs=pl.BlockSpec((rb, c), lambda i: (i, 0)),
        out_shape=jax.ShapeDtypeStruct((n, c), jnp.float32),
    )(h, gamma_col, beta_row, wc, bc_row)


def _make_gather(e_total, width, kchunk=80):
    epw = e_total // _NW
    nchunks = epw // kchunk
    mesh = plsc.VectorSubcoreMesh(core_axis_name="c", subcore_axis_name="s")

    @functools.partial(
        pl.kernel, mesh=mesh,
        out_type=[
            jax.ShapeDtypeStruct((e_total, width), jnp.int32),
            jax.ShapeDtypeStruct((e_total, width), jnp.int32),
        ],
        scratch_types=(
            [pltpu.VMEM((kchunk,), jnp.int32)] * 6
            + [pltpu.VMEM((kchunk, width), jnp.int32)] * 6
            + [pltpu.SemaphoreType.DMA] * 9
        ),
    )
    def gather_k(tbl_d, tbl_s, dst, src, out_d, out_s,
                 idx_d0, idx_s0, idx_d1, idx_s1, idx_d2, idx_s2,
                 buf_d0, buf_s0, buf_d1, buf_s1, buf_d2, buf_s2,
                 sem_d0, sem_s0, sem_d1, sem_s1, sem_d2, sem_s2,
                 sem_w0, sem_w1, sem_w2):
        wid = lax.axis_index("s") * _NC + lax.axis_index("c")
        base = wid * epw
        slots = ((idx_d0, idx_s0, buf_d0, buf_s0, sem_d0, sem_s0, sem_w0),
                 (idx_d1, idx_s1, buf_d1, buf_s1, sem_d1, sem_s1, sem_w1),
                 (idx_d2, idx_s2, buf_d2, buf_s2, sem_d2, sem_s2, sem_w2))

        def load_and_fire(ci, slot):
            idx_d, idx_s, buf_d, buf_s, sem_d, sem_s, _ = slot
            eb = base + ci * kchunk
            pltpu.sync_copy(dst.at[pl.ds(eb, kchunk)], idx_d)
            pltpu.sync_copy(src.at[pl.ds(eb, kchunk)], idx_s)
            pltpu.async_copy(tbl_d.at[idx_d], buf_d, sem_d)
            pltpu.async_copy(tbl_s.at[idx_s], buf_s, sem_s)

        def wait_writes(ci, slot):
            _, _, buf_d, buf_s, _, _, sem_w = slot
            eb = base + ci * kchunk
            pltpu.make_async_copy(buf_d, out_d.at[pl.ds(eb, kchunk)],
                                  sem_w).wait()
            pltpu.make_async_copy(buf_s, out_s.at[pl.ds(eb, kchunk)],
                                  sem_w).wait()

        load_and_fire(0, slots[0])
        load_and_fire(1, slots[1])

        def body(ci, carry):
            def per_slot(cur, nxt2):
                idx_d, idx_s, buf_d, buf_s, sem_d, sem_s, sem_w = cur
                eb = base + ci * kchunk
                pltpu.make_async_copy(tbl_d.at[idx_d], buf_d, sem_d).wait()
                pltpu.make_async_copy(tbl_s.at[idx_s], buf_s, sem_s).wait()
                pltpu.async_copy(buf_d, out_d.at[pl.ds(eb, kchunk)], sem_w)
                pltpu.async_copy(buf_s, out_s.at[pl.ds(eb, kchunk)], sem_w)

                @pl.when(ci + 2 < nchunks)
                def _():
                    @pl.when(ci >= 1)
                    def _():
                        wait_writes(ci - 1, nxt2)

                    load_and_fire(ci + 2, nxt2)

            for p in range(3):
                @pl.when(lax.rem(ci, 3) == p)
                def _(p=p):
                    per_slot(slots[p], slots[(p + 2) % 3])

            return carry

        lax.fori_loop(0, nchunks, body, 0)
        for ci in range(max(nchunks - 3, 0), nchunks):
            wait_writes(ci, slots[ci % 3])

    return gather_k


def _make_scatter_esplit(e_total, h_out, kchunk=80):
    epw = e_total // _NW
    nchunks = epw // kchunk
    rows_pt = _NPAD // _NS
    mesh = plsc.VectorSubcoreMesh(core_axis_name="c", subcore_axis_name="s")

    @functools.partial(
        pl.kernel, mesh=mesh,
        out_type=[
            jax.ShapeDtypeStruct((_NPAD, h_out), jnp.float32),
            jax.ShapeDtypeStruct((_NPAD, h_out), jnp.float32),
        ],
        scratch_types=[
            pltpu.VMEM((kchunk,), jnp.int32),
            pltpu.VMEM((kchunk,), jnp.int32),
            pltpu.VMEM((kchunk, h_out), jnp.float32),
            pltpu.VMEM((kchunk, h_out), jnp.float32),
            pltpu.VMEM_SHARED((_NPAD, h_out), jnp.float32),
            pltpu.SemaphoreType.DMA,
            pltpu.SemaphoreType.DMA,
        ],
    )
    def scatter_k(m, dstn, t_half, out0, out1,
                  idx0, idx1, buf0, buf1, acc, sem0, sem1):
        cid = lax.axis_index("c")
        sid = lax.axis_index("s")
        r0 = sid * rows_pt
        pltpu.sync_copy(t_half.at[pl.ds(r0, rows_pt)],
                        acc.at[pl.ds(r0, rows_pt)])
        plsc.subcore_barrier()
        base = (cid * _NS + sid) * epw
        slots = ((idx0, buf0, sem0), (idx1, buf1, sem1))

        def load(ci, slot):
            idx_v, buf, sem = slot
            eb = base + ci * kchunk
            pltpu.async_copy(dstn.at[pl.ds(eb, kchunk)], idx_v, sem)
            pltpu.async_copy(m.at[pl.ds(eb, kchunk)], buf, sem)

        load(0, slots[0])

        def chunk(ci, carry):
            def per_slot(cur, nxt):
                idx_v, buf, sem = cur
                eb = base + ci * kchunk
                pltpu.make_async_copy(dstn.at[pl.ds(eb, kchunk)], idx_v,
                                      sem).wait()
                pltpu.make_async_copy(m.at[pl.ds(eb, kchunk)], buf,
                                      sem).wait()

                @pl.when(ci + 1 < nchunks)
                def _():
                    load(ci + 1, nxt)

                pltpu.sync_copy(buf, acc.at[idx_v], add=True)

            @pl.when(lax.rem(ci, 2) == 0)
            def _():
                per_slot(slots[0], slots[1])

            @pl.when(lax.rem(ci, 2) == 1)
            def _():
                per_slot(slots[1], slots[0])

            return carry

        lax.fori_loop(0, nchunks, chunk, 0)
        plsc.subcore_barrier()

        @pl.when(cid == 0)
        def _():
            pltpu.sync_copy(acc.at[pl.ds(r0, rows_pt)],
                            out0.at[pl.ds(r0, rows_pt)])

        @pl.when(cid == 1)
        def _():
            pltpu.sync_copy(acc.at[pl.ds(r0, rows_pt)],
                            out1.at[pl.ds(r0, rows_pt)])

    return scatter_k


def _make_scatter_csplit(e_total, h_out, kchunk=80):
    hh = h_out // _NC
    epw = e_total // _NS
    nchunks = epw // kchunk
    rows_pt = _NPAD // _NS
    mesh = plsc.VectorSubcoreMesh(core_axis_name="c", subcore_axis_name="s")

    @functools.partial(
        pl.kernel, mesh=mesh,
        out_type=jax.ShapeDtypeStruct((_NPAD, h_out), jnp.float32),
        scratch_types=[
            pltpu.VMEM((kchunk,), jnp.int32),
            pltpu.VMEM((kchunk,), jnp.int32),
            pltpu.VMEM((kchunk, hh), jnp.float32),
            pltpu.VMEM((kchunk, hh), jnp.float32),
            pltpu.VMEM_SHARED((_NPAD, hh), jnp.float32),
            pltpu.SemaphoreType.DMA,
            pltpu.SemaphoreType.DMA,
        ],
    )
    def scatter_k(m, dstn, t_init, out,
                  idx0, idx1, buf0, buf1, acc, sem0, sem1):
        cid = lax.axis_index("c")
        sid = lax.axis_index("s")
        col0 = cid * hh
        r0 = sid * rows_pt
        pltpu.sync_copy(t_init.at[pl.ds(r0, rows_pt), pl.ds(col0, hh)],
                        acc.at[pl.ds(r0, rows_pt)])
        plsc.subcore_barrier()
        base = sid * epw
        slots = ((idx0, buf0, sem0), (idx1, buf1, sem1))

        def load(ci, slot):
            idx_v, buf, sem = slot
            eb = base + ci * kchunk
            pltpu.async_copy(dstn.at[pl.ds(eb, kchunk)], idx_v, sem)
            pltpu.async_copy(m.at[pl.ds(eb, kchunk), pl.ds(col0, hh)], buf,
                             sem)

        load(0, slots[0])

        def chunk(ci, carry):
            def per_slot(cur, nxt):
                idx_v, buf, sem = cur
                eb = base + ci * kchunk
                pltpu.make_async_copy(dstn.at[pl.ds(eb, kchunk)], idx_v,
                                      sem).wait()
                pltpu.make_async_copy(m.at[pl.ds(eb, kchunk),
                                           pl.ds(col0, hh)], buf, sem).wait()

                @pl.when(ci + 1 < nchunks)
                def _():
                    load(ci + 1, nxt)

                pltpu.sync_copy(buf, acc.at[idx_v], add=True)

            @pl.when(lax.rem(ci, 2) == 0)
            def _():
                per_slot(slots[0], slots[1])

            @pl.when(lax.rem(ci, 2) == 1)
            def _():
                per_slot(slots[1], slots[0])

            return carry

        lax.fori_loop(0, nchunks, chunk, 0)
        plsc.subcore_barrier()
        pltpu.sync_copy(acc.at[pl.ds(r0, rows_pt)],
                        out.at[pl.ds(r0, rows_pt), pl.ds(col0, hh)])

    return scatter_k


def _cgconv_layer(hins, dst, src, edge_attr, Wf, bf, Ws, bs, Wt, bt, act,
                  colsplit):
    fin = hins[0].shape[1]
    e = dst.shape[0]
    h = Wt.shape[1]
    perm = jnp.stack([jnp.arange(h), h + jnp.arange(h)], axis=1).reshape(-1)
    wd_ = jnp.concatenate([Wf[:fin], Ws[:fin]], axis=1)[:, perm]
    ws_ = jnp.concatenate([Wf[fin:2 * fin], Ws[fin:2 * fin]], axis=1)[:, perm]
    wcat = jnp.concatenate([wd_, ws_, Wt], axis=1)
    wfe = Wf[2 * fin:]
    wse = Ws[2 * fin:]

    tbl_d, tbl_s, t_self = _node_proj(hins, wcat, bt.reshape(1, h), act,
                                      wd=2 * h, wt=h, thalf=not colsplit)
    t_pad = jnp.pad(t_self, ((0, _NPAD - t_self.shape[0]), (0, 0)))
    n = tbl_d.shape[0]
    tbl_d = jax.lax.bitcast_convert_type(tbl_d.reshape(n, h, 2), jnp.int32)
    tbl_s = jax.lax.bitcast_convert_type(tbl_s.reshape(n, h, 2), jnp.int32)
    gd, gs = _make_gather(e, h, kchunk=80)(tbl_d, tbl_s, dst, src)
    m = _gate(gd, gs, edge_attr, wfe, wse, bf.reshape(1, h), bs.reshape(1, h),
              h)
    if colsplit:
        return (_make_scatter_csplit(e, h)(m, dst, t_pad),)
    return _make_scatter_esplit(e, h)(m, dst, t_pad)


def kernel(x, edge_index, edge_attr,
           Wf1, bf1, Ws1, bs1, Wt1, bt1,
           Wf2, bf2, Ws2, bs2, Wt2, bt2,
           Wf3, bf3, Ws3, bs3, Wt3, bt3,
           bn_gamma, bn_beta, Wc, bc):
    src = edge_index[0]
    dst = edge_index[1]
    h1 = _cgconv_layer((x,), dst, src, edge_attr,
                       Wf1, bf1, Ws1, bs1, Wt1, bt1,
                       act=False, colsplit=False)
    h2 = _cgconv_layer(h1, dst, src, edge_attr,
                       Wf2, bf2, Ws2, bs2, Wt2, bt2,
                       act=True, colsplit=False)
    h3 = _cgconv_layer(h2, dst, src, edge_attr,
                       Wf3, bf3, Ws3, bs3, Wt3, bt3,
                       act=True, colsplit=True)[0]
    o = h3.shape[1]
    c = Wc.shape[1]
    return _classifier(h3, bn_gamma.reshape(o, 1), bn_beta.reshape(1, o),
                       Wc, bc.reshape(1, c))
